# baseline (speedup 1.0000x reference)
import functools

import jax
import jax.numpy as jnp
from jax import lax
from jax.experimental import pallas as pl
from jax.experimental.pallas import tpu as pltpu
from jax.experimental.pallas import tpu_sc as plsc

_D = 32
_BATCH = 16384
_F = 26
_NW = 32
_BPW = _BATCH // _NW
_NH = 4
_HB = _BPW // _NH
_L = 16

_mesh = plsc.VectorSubcoreMesh(core_axis_name="c", subcore_axis_name="s")


@functools.partial(
    pl.kernel,
    mesh=_mesh,
    out_type=jax.ShapeDtypeStruct((_F, _D, _BATCH), jnp.float32),
    scratch_types=[
        pltpu.VMEM((_F, _BPW), jnp.int32),
        pltpu.VMEM((_F, _NH, _HB), jnp.int32),
        pltpu.VMEM((_HB, 128), jnp.float32),
        pltpu.VMEM((_HB, 128), jnp.float32),
        pltpu.VMEM((_D, _HB), jnp.float32),
        pltpu.VMEM((_D, _HB), jnp.float32),
        pltpu.SemaphoreType.DMA,
        pltpu.SemaphoreType.DMA,
        pltpu.SemaphoreType.DMA,
        pltpu.SemaphoreType.DMA,
    ],
    compiler_params=pltpu.CompilerParams(needs_layout_passes=False),
)
def _sc_gather(xt_hbm, srt_hbm, table_hbm, out_hbm,
               xt_v, srt_v, buf0, buf1, ot0, ot1, g0, g1, s0, s1):
    wid = lax.axis_index("s") * 2 + lax.axis_index("c")
    b0 = pl.multiple_of(wid * _BPW, _BPW)
    pltpu.sync_copy(xt_hbm.at[:, pl.ds(b0, _BPW)], xt_v)
    pltpu.sync_copy(srt_hbm.at[:, pl.ds(wid * _NH, _NH), :], srt_v)

    bufs, ots, gsem, ssem = (buf0, buf1), (ot0, ot1), (g0, g1), (s0, s1)

    def gather(f, h, b):
        return pltpu.async_copy(
            table_hbm.at[srt_v.at[f, h]], bufs[b], gsem[b])

    def extract(f, h, b):
        buf, ot = bufs[b], ots[b]

        def grp(g, carry):
            off = pl.multiple_of(h * _HB + g * _L, _L)
            v = xt_v[f, pl.ds(off, _L)]
            src_col0 = (v & 3) * _D
            lrow = lax.iota(jnp.int32, _L) + g * _L
            for c in range(_D):
                vals = plsc.load_gather(buf, [lrow, src_col0 + c])
                ot[c, pl.ds(g * _L, _L)] = vals
            return carry

        lax.fori_loop(0, _HB // _L, grp, 0)
        off = pl.multiple_of(b0 + h * _HB, _HB)
        return pltpu.async_copy(
            ot, out_hbm.at[f, :, pl.ds(off, _HB)], ssem[b])

    def field(f, carry):
        del carry
        gh = [None] * _NH
        sh = [None] * _NH
        gh[0] = gather(f, 0, 0)
        gh[1] = gather(f, 1, 1)
        for h in range(_NH):
            b = h % 2
            gh[h].wait()
            if h >= 2:
                sh[h - 2].wait()
            sh[h] = extract(f, h, b)
            if h + 2 < _NH:
                gh[h + 2] = gather(f, h + 2, b)
        sh[_NH - 2].wait()
        sh[_NH - 1].wait()
        return 0

    lax.fori_loop(0, _F, field, 0)


def kernel(x, embedding):
    xt = x.T.astype(jnp.int32)
    srt = (xt >> 2).reshape(_F, _BATCH // _HB, _HB)
    table128 = embedding.reshape(250000, 128)
    out_t = _sc_gather(xt, srt, table128)
    return jnp.transpose(out_t, (2, 0, 1))


# speedup vs baseline: 1.1820x; 1.1820x over previous
import functools

import jax
import jax.numpy as jnp
from jax import lax
from jax.experimental import pallas as pl
from jax.experimental.pallas import tpu as pltpu
from jax.experimental.pallas import tpu_sc as plsc

_D = 32
_BATCH = 16384
_F = 26
_NW = 32
_BPW = _BATCH // _NW
_NH = 4
_HB = _BPW // _NH
_L = 16

_mesh = plsc.VectorSubcoreMesh(core_axis_name="c", subcore_axis_name="s")


@functools.partial(
    pl.kernel,
    mesh=_mesh,
    out_type=jax.ShapeDtypeStruct((_F, _D, _BATCH), jnp.float32),
    scratch_types=[
        pltpu.VMEM((_F, _BPW), jnp.int32),
        pltpu.VMEM((_F, _NH, _HB), jnp.int32),
        pltpu.VMEM((_HB, 128), jnp.float32),
        pltpu.VMEM((_HB, 128), jnp.float32),
        pltpu.VMEM((_D, _HB), jnp.float32),
        pltpu.VMEM((_D, _HB), jnp.float32),
        pltpu.SemaphoreType.DMA,
        pltpu.SemaphoreType.DMA,
        pltpu.SemaphoreType.DMA,
        pltpu.SemaphoreType.DMA,
    ],
    compiler_params=pltpu.CompilerParams(needs_layout_passes=False),
)
def _sc_gather(xt_hbm, srt_hbm, table_hbm, out_hbm,
               xt_v, srt_v, buf0, buf1, ot0, ot1, g0, g1, s0, s1):
    wid = lax.axis_index("s") * 2 + lax.axis_index("c")
    b0 = pl.multiple_of(wid * _BPW, _BPW)
    pltpu.sync_copy(xt_hbm.at[:, pl.ds(b0, _BPW)], xt_v)
    pltpu.sync_copy(srt_hbm.at[:, pl.ds(wid * _NH, _NH), :], srt_v)

    bufs, ots, gsem, ssem = (buf0, buf1), (ot0, ot1), (g0, g1), (s0, s1)

    def gather(f, h, b):
        return pltpu.async_copy(
            table_hbm.at[srt_v.at[f, h]], bufs[b], gsem[b])

    def extract(f, h, b):
        buf, ot = bufs[b], ots[b]

        def grp(g, carry):
            off = pl.multiple_of(h * _HB + g * _L, _L)
            off2 = pl.multiple_of(g * _L, _L)
            v = xt_v[f, pl.ds(off, _L)]
            src_col0 = (v & 3) * _D
            lrow = lax.iota(jnp.int32, _L) + g * _L
            vals = [plsc.load_gather(buf, [lrow, src_col0 + c])
                    for c in range(_D)]
            for c in range(_D):
                ot[c, pl.ds(off2, _L)] = vals[c]
            return carry

        lax.fori_loop(0, _HB // _L, grp, 0)
        off = pl.multiple_of(b0 + h * _HB, _HB)
        return pltpu.async_copy(
            ot, out_hbm.at[f, :, pl.ds(off, _HB)], ssem[b])

    def field(f, carry):
        del carry
        gh = [None] * _NH
        sh = [None] * _NH
        gh[0] = gather(f, 0, 0)
        gh[1] = gather(f, 1, 1)
        for h in range(_NH):
            b = h % 2
            gh[h].wait()
            if h >= 2:
                sh[h - 2].wait()
            sh[h] = extract(f, h, b)
            if h + 2 < _NH:
                gh[h + 2] = gather(f, h + 2, b)
        sh[_NH - 2].wait()
        sh[_NH - 1].wait()
        return 0

    lax.fori_loop(0, _F, field, 0)


def kernel(x, embedding):
    xt = x.T.astype(jnp.int32)
    srt = (xt >> 2).reshape(_F, _BATCH // _HB, _HB)
    table128 = embedding.reshape(250000, 128)
    out_t = _sc_gather(xt, srt, table128)
    return jnp.transpose(out_t, (2, 0, 1))


# speedup vs baseline: 1.2001x; 1.0153x over previous
import functools

import jax
import jax.numpy as jnp
from jax import lax
from jax.experimental import pallas as pl
from jax.experimental.pallas import tpu as pltpu
from jax.experimental.pallas import tpu_sc as plsc

_D = 32
_B = 16384 * 26
_NC, _NS = 2, 16
_NW = _NC * _NS
_BPW = _B // _NW
_CH = 1664
_NCH = _BPW // _CH

_mesh = plsc.VectorSubcoreMesh(core_axis_name="c", subcore_axis_name="s")


@functools.partial(
    pl.kernel,
    mesh=_mesh,
    out_type=jax.ShapeDtypeStruct((_B, _D), jnp.float32),
    scratch_types=[
        pltpu.VMEM((_BPW,), jnp.int32),
        pltpu.VMEM((_CH, _D), jnp.float32),
        pltpu.VMEM((_CH, _D), jnp.float32),
        pltpu.SemaphoreType.DMA,
        pltpu.SemaphoreType.DMA,
        pltpu.SemaphoreType.DMA,
        pltpu.SemaphoreType.DMA,
    ],
    compiler_params=pltpu.CompilerParams(use_tc_tiling_on_sc=False),
)
def _sc_gather(idx_hbm, table_hbm, out_hbm, idx_v, rows0, rows1,
               g0, g1, s0, s1):
    wid = lax.axis_index("s") * _NC + lax.axis_index("c")
    base = wid * _BPW
    pltpu.sync_copy(idx_hbm.at[pl.ds(base, _BPW)], idx_v)

    rows, gsem, ssem = (rows0, rows1), (g0, g1), (s0, s1)
    gh = [None] * _NCH
    sh = [None] * _NCH
    gh[0] = pltpu.async_copy(
        table_hbm.at[idx_v.at[pl.ds(0, _CH)]], rows[0], gsem[0])
    for i in range(_NCH):
        b = i % 2
        if i + 1 < _NCH:
            nb = (i + 1) % 2
            if i >= 1:
                sh[i - 1].wait()
            gh[i + 1] = pltpu.async_copy(
                table_hbm.at[idx_v.at[pl.ds((i + 1) * _CH, _CH)]],
                rows[nb], gsem[nb])
        gh[i].wait()
        sh[i] = pltpu.async_copy(
            rows[b], out_hbm.at[pl.ds(base + i * _CH, _CH)], ssem[b])
    sh[_NCH - 2].wait()
    sh[_NCH - 1].wait()


def kernel(x, embedding):
    idx = x.T.reshape(-1).astype(jnp.int32)
    out = _sc_gather(idx, embedding)
    return out.reshape(x.shape[1], x.shape[0], _D).transpose(1, 0, 2)
